# baseline (speedup 1.0000x reference)
import functools

import jax
import jax.numpy as jnp
import numpy as np
from jax import lax
from jax.experimental import pallas as pl
from jax.experimental.pallas import tpu as pltpu
from jax.experimental.pallas import tpu_sc as plsc

S = 8192
D = 2048
E = 16
CAP = 512
TB = 1024
NBLK = S // TB
NV = S // 16
NVC = CAP // 16


def _fixed_uniform() -> np.ndarray:
    def rounds(x0, x1, k1, k2):
        rot_a = (13, 15, 26, 6)
        rot_b = (17, 29, 16, 24)
        ks = (k1, k2, np.uint32(k1 ^ k2 ^ np.uint32(0x1BD11BDA)))
        x0 = (x0 + ks[0]).astype(np.uint32)
        x1 = (x1 + ks[1]).astype(np.uint32)
        sched = ((rot_a, 1, 2, 1), (rot_b, 2, 0, 2), (rot_a, 0, 1, 3),
                 (rot_b, 1, 2, 4), (rot_a, 2, 0, 5))
        for rots, a, b, i in sched:
            for r in rots:
                x0 = (x0 + x1).astype(np.uint32)
                x1 = ((x1 << np.uint32(r))
                      | (x1 >> np.uint32(32 - r))).astype(np.uint32)
                x1 = x1 ^ x0
            x0 = (x0 + ks[a]).astype(np.uint32)
            x1 = (x1 + ks[b] + np.uint32(i)).astype(np.uint32)
        return x0, x1

    idx = np.arange(S * E, dtype=np.uint64)
    c1 = (idx >> np.uint64(32)).astype(np.uint32)
    c2 = (idx & np.uint64(0xFFFFFFFF)).astype(np.uint32)
    b1, b2 = rounds(c1, c2, np.uint32(0), np.uint32(42))
    fb = ((b1 ^ b2) >> np.uint32(9)) | np.uint32(0x3F800000)
    return (fb.view(np.float32) - np.float32(1.0)).reshape(S, E)


_U_CONST = _fixed_uniform()


def _tc1_body(x_ref, wg_ref, u_ref, gatesT_ref, randT_ref, rand_ref,
              mask1_ref, laux_ref, acc_ref):
    i = pl.program_id(0)

    logits = jnp.dot(x_ref[...], wg_ref[...],
                     preferred_element_type=jnp.float32)
    m = jnp.max(logits, axis=1, keepdims=True)
    ex = jnp.exp(logits - m)
    gates = ex / jnp.sum(ex, axis=1, keepdims=True)
    gatesT_ref[...] = gates.T

    lane = lax.broadcasted_iota(jnp.int32, (TB, E), 1)
    mx = jnp.max(gates, axis=1, keepdims=True)
    ismax = gates == mx
    firstmax = jnp.min(jnp.where(ismax, lane, E), axis=1, keepdims=True)
    mask1 = (lane == firstmax).astype(jnp.int32)
    mask1_ref[...] = mask1

    rand = mask1.astype(jnp.float32) * u_ref[...]
    rand_ref[...] = rand
    randT_ref[...] = rand.T

    @pl.when(i == 0)
    def _init():
        acc_ref[...] = jnp.zeros_like(acc_ref)
        laux_ref[...] = jnp.zeros_like(laux_ref)

    me_part = jnp.sum(gates, axis=0, keepdims=True)
    ce_part = jnp.sum(mask1.astype(jnp.float32), axis=0, keepdims=True)
    acc_ref[0:1, 0:E] += me_part
    acc_ref[1:2, 0:E] += ce_part

    @pl.when(i == NBLK - 1)
    def _fin():
        me = acc_ref[0:1, 0:E] / float(S)
        ce = acc_ref[1:2, 0:E] / float(S)
        val = jnp.sum(me * ce) * float(E)
        r0 = lax.broadcasted_iota(jnp.int32, (8, 128), 0) == 0
        c0 = lax.broadcasted_iota(jnp.int32, (8, 128), 1) == 0
        laux_ref[...] = jnp.where(r0 & c0, val, 0.0)


def _tc1(x, wg, u):
    return pl.pallas_call(
        _tc1_body,
        grid=(NBLK,),
        in_specs=[
            pl.BlockSpec((TB, D), lambda i: (i, 0)),
            pl.BlockSpec((D, E), lambda i: (0, 0)),
            pl.BlockSpec((TB, E), lambda i: (i, 0)),
        ],
        out_specs=[
            pl.BlockSpec((E, TB), lambda i: (0, i)),
            pl.BlockSpec((E, TB), lambda i: (0, i)),
            pl.BlockSpec((TB, E), lambda i: (i, 0)),
            pl.BlockSpec((TB, E), lambda i: (i, 0)),
            pl.BlockSpec((8, 128), lambda i: (0, 0)),
        ],
        out_shape=[
            jax.ShapeDtypeStruct((E, S), jnp.float32),
            jax.ShapeDtypeStruct((E, S), jnp.float32),
            jax.ShapeDtypeStruct((S, E), jnp.float32),
            jax.ShapeDtypeStruct((S, E), jnp.int32),
            jax.ShapeDtypeStruct((8, 128), jnp.float32),
        ],
        scratch_shapes=[pltpu.VMEM((8, 128), jnp.float32)],
    )(x, wg, u)


def _sc_body(gates_hbm, rand_hbm, gatings_hbm, indices_hbm, tmat_hbm,
             imat_hbm, valsf_v, bits_v, svals_v, sidx_v, svals2_v, sidx2_v,
             outf_v, row16f_v, row16i_v, hist_v, suf_v):
    e = lax.axis_index("s")
    a = lax.axis_index("c")

    iota16 = lax.iota(jnp.int32, 16)
    ones16 = jnp.ones((16,), jnp.int32)
    mn = jnp.min(plsc.cumsum(ones16))

    @pl.when(a == 0)
    def _load_g():
        pltpu.sync_copy(gates_hbm.at[pl.ds(e * S, S)], valsf_v)

    @pl.when(a == 1)
    def _load_r():
        pltpu.sync_copy(rand_hbm.at[pl.ds(e * S, S)], valsf_v)

    @plsc.parallel_loop(0, NV, unroll=8)
    def _conv(i):
        bits_v[pl.ds(i * 16, 16)] = plsc.bitcast(valsf_v[pl.ds(i * 16, 16)],
                                                 jnp.int32)

    lane_off = iota16 * 256
    p = jnp.int32(0)
    rr = jnp.int32(CAP)
    for (sh, nb) in ((23, 256), (15, 256), (7, 256), (0, 128)):
        nvreg = nb // 16

        @plsc.parallel_loop(0, 16 * (256 // 16), unroll=8)
        def _zero(i):
            hist_v[pl.ds(i * 16, 16)] = jnp.zeros((16,), jnp.int32)

        himask = jnp.int32(-(1 << (sh + (8 if nb == 256 else 7))))
        pcur = p

        def _hbody(i0, c, _sh=sh, _nb=nb, _hm=himask, _p=pcur):
            for u in range(8):
                i = i0 * 8 + u
                b = bits_v[pl.ds(i * 16, 16)]
                buck = (b >> _sh) & (_nb - 1)
                if _sh == 23:
                    plsc.addupdate_scatter(hist_v, [lane_off + buck], ones16)
                else:
                    m = (b & _hm) == _p
                    plsc.addupdate_scatter(hist_v, [lane_off + buck], ones16,
                                           mask=m)
            return c
        lax.fori_loop(0, NV // 8, _hbody, 0)

        carry = jnp.int32(0)
        bsel = jnp.int32(nb)
        for j in range(nvreg - 1, -1, -1):
            acc = jnp.zeros((16,), jnp.int32)
            for l in range(16):
                acc = acc + hist_v[pl.ds(l * 256 + j * 16, 16)]
            rv = lax.rev(acc, (0,))
            cs = plsc.cumsum(rv)
            incl = lax.rev(cs + (1 - mn) * rv, (0,))
            suf_j = incl - acc + carry
            suf_v[pl.ds(j * 16, 16)] = suf_j
            candv = jnp.where(suf_j < rr, iota16 + j * 16, nb)
            bsel = jnp.minimum(bsel, jnp.min(candv))
            carry = carry + jnp.sum(acc)
        rr = rr - suf_v[pl.ds(bsel, 16)][0]
        p = p | (bsel << sh)
    t_bits = p
    r = rr

    @plsc.parallel_loop(
        0, NV, unroll=4,
        carry=(jnp.int32(0), jnp.int32(0), jnp.full((16,), -1, jnp.int32)))
    def _selpass(i, carry):
        cnt_eq, off, imax = carry
        b = bits_v[pl.ds(i * 16, 16)]
        gt = b > t_bits
        eq = b == t_bits
        eqi = jnp.where(eq, 1, 0)
        excl_eq = plsc.cumsum(eqi) - mn * eqi
        tie = eq & ((cnt_eq + excl_eq) < r)
        sel = gt | tie
        seli = jnp.where(sel, 1, 0)
        idxv = iota16 + i * 16
        imax = jnp.maximum(imax, jnp.where(tie, idxv, -1))
        tgt = off + (plsc.cumsum(seli) - mn * seli)
        plsc.store_scatter(svals_v, [tgt], b, mask=sel)
        plsc.store_scatter(sidx_v, [tgt], idxv, mask=sel)
        return (cnt_eq + jnp.sum(eqi), off + jnp.sum(seli), imax)

    _, _, imax = _selpass
    i_cut = jnp.max(imax)

    @pl.when(a == 1)
    def _emit_thr():
        row16f_v[...] = plsc.bitcast(jnp.full((16,), t_bits, jnp.int32),
                                     jnp.float32)
        row16i_v[...] = jnp.full((16,), i_cut, jnp.int32)
        pltpu.sync_copy(row16f_v, tmat_hbm.at[pl.ds(e * 16, 16)])
        pltpu.sync_copy(row16i_v, imat_hbm.at[pl.ds(e * 16, 16)])

    @pl.when(a == 0)
    def _sort():
        bufs = ((svals_v, sidx_v), (svals2_v, sidx2_v))
        stage = 0
        for kk in (2, 4, 8, 16, 32, 64, 128, 256, 512):
            jj = kk // 2
            while jj >= 1:
                src_k, src_i = bufs[stage % 2]
                dst_k, dst_i = bufs[1 - stage % 2]

                @plsc.parallel_loop(0, NVC, unroll=2)
                def sub(v, _j=jj, _k=kk, _sk=src_k, _si=src_i,
                        _dk=dst_k, _di=dst_i):
                    gidx = jnp.full((16,), v * 16, jnp.int32) + iota16
                    pidx = gidx ^ _j
                    kv = _sk[pl.ds(v * 16, 16)]
                    iv = _si[pl.ds(v * 16, 16)]
                    kp = plsc.load_gather(_sk, [pidx])
                    ip = plsc.load_gather(_si, [pidx])
                    first = (kv > kp) | ((kv == kp) & (iv < ip))
                    is_low = (gidx & _j) == 0
                    dirasc = (gidx & _k) == 0
                    keep = first == (is_low == dirasc)
                    _dk[pl.ds(v * 16, 16)] = jnp.where(keep, kv, kp)
                    _di[pl.ds(v * 16, 16)] = jnp.where(keep, iv, ip)

                stage += 1
                jj //= 2
        fin_k, fin_i = bufs[stage % 2]

        @plsc.parallel_loop(0, NVC, unroll=8)
        def conv(i):
            outf_v[pl.ds(i * 16, 16)] = plsc.bitcast(
                fin_k[pl.ds(i * 16, 16)], jnp.float32)
        pltpu.sync_copy(outf_v, gatings_hbm.at[pl.ds(e * CAP, CAP)])
        pltpu.sync_copy(fin_i, indices_hbm.at[pl.ds(e * CAP, CAP)])


def _sc_select(gates, rand):
    mesh = plsc.VectorSubcoreMesh(core_axis_name="c", subcore_axis_name="s")
    fn = pl.kernel(
        _sc_body,
        mesh=mesh,
        compiler_params=pltpu.CompilerParams(needs_layout_passes=False),
        out_type=[
            jax.ShapeDtypeStruct((E * CAP,), jnp.float32),
            jax.ShapeDtypeStruct((E * CAP,), jnp.int32),
            jax.ShapeDtypeStruct((E * 16,), jnp.float32),
            jax.ShapeDtypeStruct((E * 16,), jnp.int32),
        ],
        scratch_types=[
            pltpu.VMEM((S,), jnp.float32),
            pltpu.VMEM((S,), jnp.int32),
            pltpu.VMEM((CAP + 16,), jnp.int32),
            pltpu.VMEM((CAP + 16,), jnp.int32),
            pltpu.VMEM((CAP,), jnp.int32),
            pltpu.VMEM((CAP,), jnp.int32),
            pltpu.VMEM((CAP,), jnp.float32),
            pltpu.VMEM((16,), jnp.float32),
            pltpu.VMEM((16,), jnp.int32),
            pltpu.VMEM((16 * 256,), jnp.int32),
            pltpu.VMEM((256 + 16,), jnp.int32),
        ],
    )
    return fn(gates, rand)


def _tc2_body(mask1_ref, rand_ref, tmat_ref, imat_ref, out_ref):
    i = pl.program_id(0)
    ii = lax.broadcasted_iota(jnp.int32, (E, E), 0)
    jj = lax.broadcasted_iota(jnp.int32, (E, E), 1)
    eye = ii == jj
    tdiag = jnp.sum(jnp.where(eye, tmat_ref[...], 0.0), axis=0,
                    keepdims=True)
    idiag = jnp.sum(jnp.where(eye, imat_ref[...], 0), axis=0,
                    keepdims=True)
    tok = lax.broadcasted_iota(jnp.int32, (TB, E), 0) + i * TB
    rnd = rand_ref[...]
    sel = (rnd > tdiag) | ((rnd == tdiag) & (tok <= idiag))
    out_ref[...] = jnp.where(sel & (mask1_ref[...] > 0), 1, 0)


def _tc2(mask1, rand, tmat, imat):
    return pl.pallas_call(
        _tc2_body,
        grid=(NBLK,),
        in_specs=[
            pl.BlockSpec((TB, E), lambda i: (i, 0)),
            pl.BlockSpec((TB, E), lambda i: (i, 0)),
            pl.BlockSpec((E, 16), lambda i: (0, 0)),
            pl.BlockSpec((E, 16), lambda i: (0, 0)),
        ],
        out_specs=pl.BlockSpec((TB, E), lambda i: (i, 0)),
        out_shape=jax.ShapeDtypeStruct((S, E), jnp.int32),
    )(mask1, rand, tmat, imat)


def kernel(x, wg):
    u = jnp.asarray(_U_CONST)
    gatesT, randT, rand, mask1, lauxm = _tc1(x, wg, u)
    gat_f, ind_f, trow, irow = _sc_select(gatesT.reshape(-1),
                                          randT.reshape(-1))
    new_mask1 = _tc2(mask1, rand, trow.reshape(E, 16), irow.reshape(E, 16))
    return (lauxm[0, 0], gat_f.reshape(E, CAP), ind_f.reshape(E, CAP),
            new_mask1)


# speedup vs baseline: 1.1998x; 1.1998x over previous
import functools

import jax
import jax.numpy as jnp
import numpy as np
from jax import lax
from jax.experimental import pallas as pl
from jax.experimental.pallas import tpu as pltpu
from jax.experimental.pallas import tpu_sc as plsc

S = 8192
D = 2048
E = 16
CAP = 512
TB = 2048
NBLK = S // TB
NV = S // 16
NVC = CAP // 16


def _fixed_uniform() -> np.ndarray:
    def rounds(x0, x1, k1, k2):
        rot_a = (13, 15, 26, 6)
        rot_b = (17, 29, 16, 24)
        ks = (k1, k2, np.uint32(k1 ^ k2 ^ np.uint32(0x1BD11BDA)))
        x0 = (x0 + ks[0]).astype(np.uint32)
        x1 = (x1 + ks[1]).astype(np.uint32)
        sched = ((rot_a, 1, 2, 1), (rot_b, 2, 0, 2), (rot_a, 0, 1, 3),
                 (rot_b, 1, 2, 4), (rot_a, 2, 0, 5))
        for rots, a, b, i in sched:
            for r in rots:
                x0 = (x0 + x1).astype(np.uint32)
                x1 = ((x1 << np.uint32(r))
                      | (x1 >> np.uint32(32 - r))).astype(np.uint32)
                x1 = x1 ^ x0
            x0 = (x0 + ks[a]).astype(np.uint32)
            x1 = (x1 + ks[b] + np.uint32(i)).astype(np.uint32)
        return x0, x1

    idx = np.arange(S * E, dtype=np.uint64)
    c1 = (idx >> np.uint64(32)).astype(np.uint32)
    c2 = (idx & np.uint64(0xFFFFFFFF)).astype(np.uint32)
    b1, b2 = rounds(c1, c2, np.uint32(0), np.uint32(42))
    fb = ((b1 ^ b2) >> np.uint32(9)) | np.uint32(0x3F800000)
    return (fb.view(np.float32) - np.float32(1.0)).reshape(S, E)


_U_CONST = _fixed_uniform()


def _tc1_body(x_ref, wg_ref, u_ref, gatesT_ref, randT_ref, rand_ref,
              mask1_ref, laux_ref, acc_ref):
    i = pl.program_id(0)

    logits = jnp.dot(x_ref[...], wg_ref[...],
                     preferred_element_type=jnp.float32)
    m = jnp.max(logits, axis=1, keepdims=True)
    ex = jnp.exp(logits - m)
    gates = ex / jnp.sum(ex, axis=1, keepdims=True)
    gatesT_ref[...] = gates.T

    lane = lax.broadcasted_iota(jnp.int32, (TB, E), 1)
    mx = jnp.max(gates, axis=1, keepdims=True)
    ismax = gates == mx
    firstmax = jnp.min(jnp.where(ismax, lane, E), axis=1, keepdims=True)
    mask1 = (lane == firstmax).astype(jnp.int32)
    mask1_ref[...] = mask1

    rand = mask1.astype(jnp.float32) * u_ref[...]
    rand_ref[...] = rand
    randT_ref[...] = rand.T

    @pl.when(i == 0)
    def _init():
        acc_ref[...] = jnp.zeros_like(acc_ref)
        laux_ref[...] = jnp.zeros_like(laux_ref)

    me_part = jnp.sum(gates, axis=0, keepdims=True)
    ce_part = jnp.sum(mask1.astype(jnp.float32), axis=0, keepdims=True)
    acc_ref[0:1, 0:E] += me_part
    acc_ref[1:2, 0:E] += ce_part

    @pl.when(i == NBLK - 1)
    def _fin():
        me = acc_ref[0:1, 0:E] / float(S)
        ce = acc_ref[1:2, 0:E] / float(S)
        val = jnp.sum(me * ce) * float(E)
        r0 = lax.broadcasted_iota(jnp.int32, (8, 128), 0) == 0
        c0 = lax.broadcasted_iota(jnp.int32, (8, 128), 1) == 0
        laux_ref[...] = jnp.where(r0 & c0, val, 0.0)


def _tc1(x, wg, u):
    return pl.pallas_call(
        _tc1_body,
        grid=(NBLK,),
        in_specs=[
            pl.BlockSpec((TB, D), lambda i: (i, 0)),
            pl.BlockSpec((D, E), lambda i: (0, 0)),
            pl.BlockSpec((TB, E), lambda i: (i, 0)),
        ],
        out_specs=[
            pl.BlockSpec((E, TB), lambda i: (0, i)),
            pl.BlockSpec((E, TB), lambda i: (0, i)),
            pl.BlockSpec((TB, E), lambda i: (i, 0)),
            pl.BlockSpec((TB, E), lambda i: (i, 0)),
            pl.BlockSpec((8, 128), lambda i: (0, 0)),
        ],
        out_shape=[
            jax.ShapeDtypeStruct((E, S), jnp.float32),
            jax.ShapeDtypeStruct((E, S), jnp.float32),
            jax.ShapeDtypeStruct((S, E), jnp.float32),
            jax.ShapeDtypeStruct((S, E), jnp.int32),
            jax.ShapeDtypeStruct((8, 128), jnp.float32),
        ],
        scratch_shapes=[pltpu.VMEM((8, 128), jnp.float32)],
    )(x, wg, u)


def _sc_body(gates_hbm, rand_hbm, gatings_hbm, indices_hbm, tmat_hbm,
             imat_hbm, valsf_v, bits_v, svals_v, sidx_v, svals2_v, sidx2_v,
             outf_v, row16f_v, row16i_v):
    e = lax.axis_index("s")
    a = lax.axis_index("c")

    iota16 = lax.iota(jnp.int32, 16)
    ones16 = jnp.ones((16,), jnp.int32)
    mn = jnp.min(plsc.cumsum(ones16))

    @pl.when(a == 0)
    def _load_g():
        pltpu.sync_copy(gates_hbm.at[pl.ds(e * S, S)], valsf_v)

    @pl.when(a == 1)
    def _load_r():
        pltpu.sync_copy(rand_hbm.at[pl.ds(e * S, S)], valsf_v)

    @plsc.parallel_loop(0, NV, unroll=8)
    def _conv(i):
        bits_v[pl.ds(i * 16, 16)] = plsc.bitcast(valsf_v[pl.ds(i * 16, 16)],
                                                 jnp.int32)

    def _count_gt(thr):
        @plsc.parallel_loop(0, NV, unroll=8,
                            carry=jnp.zeros((16,), jnp.int32))
        def body(i, acc):
            v = bits_v[pl.ds(i * 16, 16)]
            return acc + jnp.where(v > thr, 1, 0)
        return jnp.sum(body)

    def _bs(k, lohi):
        lo, hi = lohi
        mid = lo + ((hi - lo) >> 1)
        c = _count_gt(mid)
        lo2 = jnp.where(c < CAP, lo, mid + 1)
        hi2 = jnp.where(c < CAP, mid, hi)
        return (lo2, hi2)
    _, t_bits = lax.fori_loop(0, 31, _bs,
                              (jnp.int32(0), jnp.int32(2**31 - 1)))
    c_gt = _count_gt(t_bits)
    r = CAP - c_gt

    @plsc.parallel_loop(
        0, NV, unroll=4,
        carry=(jnp.int32(0), jnp.int32(0), jnp.full((16,), -1, jnp.int32)))
    def _selpass(i, carry):
        cnt_eq, off, imax = carry
        b = bits_v[pl.ds(i * 16, 16)]
        gt = b > t_bits
        eq = b == t_bits
        eqi = jnp.where(eq, 1, 0)
        excl_eq = plsc.cumsum(eqi) - mn * eqi
        tie = eq & ((cnt_eq + excl_eq) < r)
        sel = gt | tie
        seli = jnp.where(sel, 1, 0)
        idxv = iota16 + i * 16
        imax = jnp.maximum(imax, jnp.where(tie, idxv, -1))
        tgt = off + (plsc.cumsum(seli) - mn * seli)
        plsc.store_scatter(svals_v, [tgt], b, mask=sel)
        plsc.store_scatter(sidx_v, [tgt], idxv, mask=sel)
        return (cnt_eq + jnp.sum(eqi), off + jnp.sum(seli), imax)

    _, _, imax = _selpass
    i_cut = jnp.max(imax)

    @pl.when(a == 1)
    def _emit_thr():
        row16f_v[...] = plsc.bitcast(jnp.full((16,), t_bits, jnp.int32),
                                     jnp.float32)
        row16i_v[...] = jnp.full((16,), i_cut, jnp.int32)
        pltpu.sync_copy(row16f_v, tmat_hbm.at[pl.ds(e * 16, 16)])
        pltpu.sync_copy(row16i_v, imat_hbm.at[pl.ds(e * 16, 16)])

    @pl.when(a == 0)
    def _sort():
        bufs = ((svals_v, sidx_v), (svals2_v, sidx2_v))
        stage = 0
        for kk in (2, 4, 8, 16, 32, 64, 128, 256, 512):
            jj = kk // 2
            while jj >= 1:
                src_k, src_i = bufs[stage % 2]
                dst_k, dst_i = bufs[1 - stage % 2]

                @plsc.parallel_loop(0, NVC, unroll=2)
                def sub(v, _j=jj, _k=kk, _sk=src_k, _si=src_i,
                        _dk=dst_k, _di=dst_i):
                    gidx = jnp.full((16,), v * 16, jnp.int32) + iota16
                    pidx = gidx ^ _j
                    kv = _sk[pl.ds(v * 16, 16)]
                    iv = _si[pl.ds(v * 16, 16)]
                    kp = plsc.load_gather(_sk, [pidx])
                    ip = plsc.load_gather(_si, [pidx])
                    first = (kv > kp) | ((kv == kp) & (iv < ip))
                    is_low = (gidx & _j) == 0
                    dirasc = (gidx & _k) == 0
                    keep = first == (is_low == dirasc)
                    _dk[pl.ds(v * 16, 16)] = jnp.where(keep, kv, kp)
                    _di[pl.ds(v * 16, 16)] = jnp.where(keep, iv, ip)

                stage += 1
                jj //= 2
        fin_k, fin_i = bufs[stage % 2]

        @plsc.parallel_loop(0, NVC, unroll=8)
        def conv(i):
            outf_v[pl.ds(i * 16, 16)] = plsc.bitcast(
                fin_k[pl.ds(i * 16, 16)], jnp.float32)
        pltpu.sync_copy(outf_v, gatings_hbm.at[pl.ds(e * CAP, CAP)])
        pltpu.sync_copy(fin_i, indices_hbm.at[pl.ds(e * CAP, CAP)])


def _sc_select(gates, rand):
    mesh = plsc.VectorSubcoreMesh(core_axis_name="c", subcore_axis_name="s")
    fn = pl.kernel(
        _sc_body,
        mesh=mesh,
        compiler_params=pltpu.CompilerParams(needs_layout_passes=False),
        out_type=[
            jax.ShapeDtypeStruct((E * CAP,), jnp.float32),
            jax.ShapeDtypeStruct((E * CAP,), jnp.int32),
            jax.ShapeDtypeStruct((E * 16,), jnp.float32),
            jax.ShapeDtypeStruct((E * 16,), jnp.int32),
        ],
        scratch_types=[
            pltpu.VMEM((S,), jnp.float32),
            pltpu.VMEM((S,), jnp.int32),
            pltpu.VMEM((CAP + 16,), jnp.int32),
            pltpu.VMEM((CAP + 16,), jnp.int32),
            pltpu.VMEM((CAP,), jnp.int32),
            pltpu.VMEM((CAP,), jnp.int32),
            pltpu.VMEM((CAP,), jnp.float32),
            pltpu.VMEM((16,), jnp.float32),
            pltpu.VMEM((16,), jnp.int32),
        ],
    )
    return fn(gates, rand)


def _tc2_body(mask1_ref, rand_ref, tmat_ref, imat_ref, out_ref):
    i = pl.program_id(0)
    ii = lax.broadcasted_iota(jnp.int32, (E, E), 0)
    jj = lax.broadcasted_iota(jnp.int32, (E, E), 1)
    eye = ii == jj
    tdiag = jnp.sum(jnp.where(eye, tmat_ref[...], 0.0), axis=0,
                    keepdims=True)
    idiag = jnp.sum(jnp.where(eye, imat_ref[...], 0), axis=0,
                    keepdims=True)
    tok = lax.broadcasted_iota(jnp.int32, (TB, E), 0) + i * TB
    rnd = rand_ref[...]
    sel = (rnd > tdiag) | ((rnd == tdiag) & (tok <= idiag))
    out_ref[...] = jnp.where(sel & (mask1_ref[...] > 0), 1, 0)


def _tc2(mask1, rand, tmat, imat):
    return pl.pallas_call(
        _tc2_body,
        grid=(NBLK,),
        in_specs=[
            pl.BlockSpec((TB, E), lambda i: (i, 0)),
            pl.BlockSpec((TB, E), lambda i: (i, 0)),
            pl.BlockSpec((E, 16), lambda i: (0, 0)),
            pl.BlockSpec((E, 16), lambda i: (0, 0)),
        ],
        out_specs=pl.BlockSpec((TB, E), lambda i: (i, 0)),
        out_shape=jax.ShapeDtypeStruct((S, E), jnp.int32),
    )(mask1, rand, tmat, imat)


def kernel(x, wg):
    u = jnp.asarray(_U_CONST)
    gatesT, randT, rand, mask1, lauxm = _tc1(x, wg, u)
    gat_f, ind_f, trow, irow = _sc_select(gatesT.reshape(-1),
                                          randT.reshape(-1))
    new_mask1 = _tc2(mask1, rand, trow.reshape(E, 16), irow.reshape(E, 16))
    return (lauxm[0, 0], gat_f.reshape(E, CAP), ind_f.reshape(E, CAP),
            new_mask1)


# speedup vs baseline: 1.2256x; 1.0215x over previous
import functools

import jax
import jax.numpy as jnp
import numpy as np
from jax import lax
from jax.experimental import pallas as pl
from jax.experimental.pallas import tpu as pltpu
from jax.experimental.pallas import tpu_sc as plsc

S = 8192
D = 2048
E = 16
CAP = 512
TB = 1024
NBLK = S // TB
NV = S // 16
NVC = CAP // 16


def _fixed_uniform() -> np.ndarray:
    def rounds(x0, x1, k1, k2):
        rot_a = (13, 15, 26, 6)
        rot_b = (17, 29, 16, 24)
        ks = (k1, k2, np.uint32(k1 ^ k2 ^ np.uint32(0x1BD11BDA)))
        x0 = (x0 + ks[0]).astype(np.uint32)
        x1 = (x1 + ks[1]).astype(np.uint32)
        sched = ((rot_a, 1, 2, 1), (rot_b, 2, 0, 2), (rot_a, 0, 1, 3),
                 (rot_b, 1, 2, 4), (rot_a, 2, 0, 5))
        for rots, a, b, i in sched:
            for r in rots:
                x0 = (x0 + x1).astype(np.uint32)
                x1 = ((x1 << np.uint32(r))
                      | (x1 >> np.uint32(32 - r))).astype(np.uint32)
                x1 = x1 ^ x0
            x0 = (x0 + ks[a]).astype(np.uint32)
            x1 = (x1 + ks[b] + np.uint32(i)).astype(np.uint32)
        return x0, x1

    idx = np.arange(S * E, dtype=np.uint64)
    c1 = (idx >> np.uint64(32)).astype(np.uint32)
    c2 = (idx & np.uint64(0xFFFFFFFF)).astype(np.uint32)
    b1, b2 = rounds(c1, c2, np.uint32(0), np.uint32(42))
    fb = ((b1 ^ b2) >> np.uint32(9)) | np.uint32(0x3F800000)
    return (fb.view(np.float32) - np.float32(1.0)).reshape(S, E)


_U_CONST = _fixed_uniform()


def _tc1_body(x_ref, wg_ref, u_ref, grT_ref, rand_ref,
              mask1_ref, laux_ref, acc_ref):
    i = pl.program_id(0)

    logits = jnp.dot(x_ref[...], wg_ref[...],
                     preferred_element_type=jnp.float32)
    m = jnp.max(logits, axis=1, keepdims=True)
    ex = jnp.exp(logits - m)
    gates = ex / jnp.sum(ex, axis=1, keepdims=True)
    grT_ref[0:E, :] = gates.T

    lane = lax.broadcasted_iota(jnp.int32, (TB, E), 1)
    mx = jnp.max(gates, axis=1, keepdims=True)
    ismax = gates == mx
    firstmax = jnp.min(jnp.where(ismax, lane, E), axis=1, keepdims=True)
    mask1 = (lane == firstmax).astype(jnp.int32)
    mask1_ref[...] = mask1

    rand = mask1.astype(jnp.float32) * u_ref[...]
    rand_ref[...] = rand
    grT_ref[E:2 * E, :] = rand.T

    @pl.when(i == 0)
    def _init():
        acc_ref[...] = jnp.zeros_like(acc_ref)
        laux_ref[...] = jnp.zeros_like(laux_ref)

    me_part = jnp.sum(gates, axis=0, keepdims=True)
    ce_part = jnp.sum(mask1.astype(jnp.float32), axis=0, keepdims=True)
    acc_ref[0:1, 0:E] += me_part
    acc_ref[1:2, 0:E] += ce_part

    @pl.when(i == NBLK - 1)
    def _fin():
        me = acc_ref[0:1, 0:E] / float(S)
        ce = acc_ref[1:2, 0:E] / float(S)
        val = jnp.sum(me * ce) * float(E)
        r0 = lax.broadcasted_iota(jnp.int32, (8, 128), 0) == 0
        c0 = lax.broadcasted_iota(jnp.int32, (8, 128), 1) == 0
        laux_ref[...] = jnp.where(r0 & c0, val, 0.0)


def _tc1(x, wg, u):
    return pl.pallas_call(
        _tc1_body,
        grid=(NBLK,),
        in_specs=[
            pl.BlockSpec((TB, D), lambda i: (i, 0)),
            pl.BlockSpec((D, E), lambda i: (0, 0)),
            pl.BlockSpec((TB, E), lambda i: (i, 0)),
        ],
        out_specs=[
            pl.BlockSpec((2 * E, TB), lambda i: (0, i)),
            pl.BlockSpec((TB, E), lambda i: (i, 0)),
            pl.BlockSpec((TB, E), lambda i: (i, 0)),
            pl.BlockSpec((8, 128), lambda i: (0, 0)),
        ],
        out_shape=[
            jax.ShapeDtypeStruct((2 * E, S), jnp.float32),
            jax.ShapeDtypeStruct((S, E), jnp.float32),
            jax.ShapeDtypeStruct((S, E), jnp.int32),
            jax.ShapeDtypeStruct((8, 128), jnp.float32),
        ],
        scratch_shapes=[pltpu.VMEM((8, 128), jnp.float32)],
    )(x, wg, u)


def _sc_body(gr_hbm, gatings_hbm, indices_hbm, tmat_hbm,
             imat_hbm, valsf_v, bits_v, svals_v, sidx_v, svals2_v, sidx2_v,
             outf_v, row16f_v, row16i_v):
    e = lax.axis_index("s")
    a = lax.axis_index("c")

    iota16 = lax.iota(jnp.int32, 16)
    ones16 = jnp.ones((16,), jnp.int32)
    mn = jnp.min(plsc.cumsum(ones16))

    pltpu.sync_copy(gr_hbm.at[pl.ds((a * E + e) * S, S)], valsf_v)

    @plsc.parallel_loop(0, NV, unroll=8)
    def _conv(i):
        bits_v[pl.ds(i * 16, 16)] = plsc.bitcast(valsf_v[pl.ds(i * 16, 16)],
                                                 jnp.int32)

    def _count_gt(thr):
        @plsc.parallel_loop(0, NV, unroll=8,
                            carry=jnp.zeros((16,), jnp.int32))
        def body(i, acc):
            v = bits_v[pl.ds(i * 16, 16)]
            return acc + jnp.where(v > thr, 1, 0)
        return jnp.sum(body)

    def _bs(k, lohi):
        lo, hi = lohi
        mid = lo + ((hi - lo) >> 1)
        c = _count_gt(mid)
        lo2 = jnp.where(c < CAP, lo, mid + 1)
        hi2 = jnp.where(c < CAP, mid, hi)
        return (lo2, hi2)
    _, t_bits = lax.fori_loop(0, 31, _bs,
                              (jnp.int32(0), jnp.int32(2**31 - 1)))
    c_gt = _count_gt(t_bits)
    r = CAP - c_gt

    @plsc.parallel_loop(
        0, NV, unroll=4,
        carry=(jnp.int32(0), jnp.int32(0), jnp.full((16,), -1, jnp.int32)))
    def _selpass(i, carry):
        cnt_eq, off, imax = carry
        b = bits_v[pl.ds(i * 16, 16)]
        gt = b > t_bits
        eq = b == t_bits
        eqi = jnp.where(eq, 1, 0)
        excl_eq = plsc.cumsum(eqi) - mn * eqi
        tie = eq & ((cnt_eq + excl_eq) < r)
        sel = gt | tie
        seli = jnp.where(sel, 1, 0)
        idxv = iota16 + i * 16
        imax = jnp.maximum(imax, jnp.where(tie, idxv, -1))
        tgt = off + (plsc.cumsum(seli) - mn * seli)
        plsc.store_scatter(svals_v, [tgt], b, mask=sel)
        plsc.store_scatter(sidx_v, [tgt], idxv, mask=sel)
        return (cnt_eq + jnp.sum(eqi), off + jnp.sum(seli), imax)

    _, _, imax = _selpass
    i_cut = jnp.max(imax)

    @pl.when(a == 1)
    def _emit_thr():
        row16f_v[...] = plsc.bitcast(jnp.full((16,), t_bits, jnp.int32),
                                     jnp.float32)
        row16i_v[...] = jnp.full((16,), i_cut, jnp.int32)
        pltpu.sync_copy(row16f_v, tmat_hbm.at[pl.ds(e * 16, 16)])
        pltpu.sync_copy(row16i_v, imat_hbm.at[pl.ds(e * 16, 16)])

    @pl.when(a == 0)
    def _sort():
        bufs = ((svals_v, sidx_v), (svals2_v, sidx2_v))
        stage = 0
        for kk in (2, 4, 8, 16, 32, 64, 128, 256, 512):
            jj = kk // 2
            while jj >= 1:
                src_k, src_i = bufs[stage % 2]
                dst_k, dst_i = bufs[1 - stage % 2]

                @plsc.parallel_loop(0, NVC, unroll=2)
                def sub(v, _j=jj, _k=kk, _sk=src_k, _si=src_i,
                        _dk=dst_k, _di=dst_i):
                    gidx = jnp.full((16,), v * 16, jnp.int32) + iota16
                    pidx = gidx ^ _j
                    kv = _sk[pl.ds(v * 16, 16)]
                    iv = _si[pl.ds(v * 16, 16)]
                    kp = plsc.load_gather(_sk, [pidx])
                    ip = plsc.load_gather(_si, [pidx])
                    first = (kv > kp) | ((kv == kp) & (iv < ip))
                    is_low = (gidx & _j) == 0
                    dirasc = (gidx & _k) == 0
                    keep = first == (is_low == dirasc)
                    _dk[pl.ds(v * 16, 16)] = jnp.where(keep, kv, kp)
                    _di[pl.ds(v * 16, 16)] = jnp.where(keep, iv, ip)

                stage += 1
                jj //= 2
        fin_k, fin_i = bufs[stage % 2]

        @plsc.parallel_loop(0, NVC, unroll=8)
        def conv(i):
            outf_v[pl.ds(i * 16, 16)] = plsc.bitcast(
                fin_k[pl.ds(i * 16, 16)], jnp.float32)
        pltpu.sync_copy(outf_v, gatings_hbm.at[pl.ds(e * CAP, CAP)])
        pltpu.sync_copy(fin_i, indices_hbm.at[pl.ds(e * CAP, CAP)])


def _sc_select(gr):
    mesh = plsc.VectorSubcoreMesh(core_axis_name="c", subcore_axis_name="s")
    fn = pl.kernel(
        _sc_body,
        mesh=mesh,
        compiler_params=pltpu.CompilerParams(needs_layout_passes=False),
        out_type=[
            jax.ShapeDtypeStruct((E * CAP,), jnp.float32),
            jax.ShapeDtypeStruct((E * CAP,), jnp.int32),
            jax.ShapeDtypeStruct((E * 16,), jnp.float32),
            jax.ShapeDtypeStruct((E * 16,), jnp.int32),
        ],
        scratch_types=[
            pltpu.VMEM((S,), jnp.float32),
            pltpu.VMEM((S,), jnp.int32),
            pltpu.VMEM((CAP + 16,), jnp.int32),
            pltpu.VMEM((CAP + 16,), jnp.int32),
            pltpu.VMEM((CAP,), jnp.int32),
            pltpu.VMEM((CAP,), jnp.int32),
            pltpu.VMEM((CAP,), jnp.float32),
            pltpu.VMEM((16,), jnp.float32),
            pltpu.VMEM((16,), jnp.int32),
        ],
    )
    return fn(gr)


def _tc2_body(mask1_ref, rand_ref, tmat_ref, imat_ref, out_ref):
    i = pl.program_id(0)
    ii = lax.broadcasted_iota(jnp.int32, (E, E), 0)
    jj = lax.broadcasted_iota(jnp.int32, (E, E), 1)
    eye = ii == jj
    tdiag = jnp.sum(jnp.where(eye, tmat_ref[...], 0.0), axis=0,
                    keepdims=True)
    idiag = jnp.sum(jnp.where(eye, imat_ref[...], 0), axis=0,
                    keepdims=True)
    tok = lax.broadcasted_iota(jnp.int32, (TB, E), 0) + i * TB
    rnd = rand_ref[...]
    sel = (rnd > tdiag) | ((rnd == tdiag) & (tok <= idiag))
    out_ref[...] = jnp.where(sel & (mask1_ref[...] > 0), 1, 0)


def _tc2(mask1, rand, tmat, imat):
    return pl.pallas_call(
        _tc2_body,
        grid=(NBLK,),
        in_specs=[
            pl.BlockSpec((TB, E), lambda i: (i, 0)),
            pl.BlockSpec((TB, E), lambda i: (i, 0)),
            pl.BlockSpec((E, 16), lambda i: (0, 0)),
            pl.BlockSpec((E, 16), lambda i: (0, 0)),
        ],
        out_specs=pl.BlockSpec((TB, E), lambda i: (i, 0)),
        out_shape=jax.ShapeDtypeStruct((S, E), jnp.int32),
    )(mask1, rand, tmat, imat)


def kernel(x, wg):
    u = jnp.asarray(_U_CONST)
    grT, rand, mask1, lauxm = _tc1(x, wg, u)
    gat_f, ind_f, trow, irow = _sc_select(grT.reshape(-1))
    new_mask1 = _tc2(mask1, rand, trow.reshape(E, 16), irow.reshape(E, 16))
    return (lauxm[0, 0], gat_f.reshape(E, CAP), ind_f.reshape(E, CAP),
            new_mask1)
